# baseline (speedup 1.0000x reference)
import functools

import jax
import jax.numpy as jnp
from jax import lax
from jax.experimental import pallas as pl
from jax.experimental.pallas import tpu as pltpu
from jax.experimental.pallas import tpu_sc as plsc

N = 10000
E = 320000
IN_DIM = 128
HID = 128
F = 64
K_PROP = 10
ALPHA = 0.1
EPS = 1e-5

NC, NS = 2, 16
NW = NC * NS
NPAD = 10240
RPT = NPAD // NS
RB = 64
NBLK = RPT // RB
EPT = E // NW
CH = 64
NCHUNK = 160
NSLOT = 5
SLOTS = NCHUNK * CH
DW = 16
CHD = 128
NCHD = SLOTS // CHD

_HIGH = lax.Precision.HIGHEST


def _mesh():
    return plsc.VectorSubcoreMesh(
        core_axis_name="c", subcore_axis_name="s", num_cores=NC, num_subcores=NS
    )


def _sc_prop(name):
    scratch = (
        [
            pltpu.VMEM_SHARED((NPAD, F), jnp.float32),
            pltpu.VMEM_SHARED((NPAD, F), jnp.float32),
        ]
        + [pltpu.VMEM((2, CH), jnp.int32) for _ in range(NSLOT)]
        + [pltpu.VMEM((CH, F), jnp.float32) for _ in range(NSLOT)]
        + [pltpu.SemaphoreType.DMA] * (2 * NSLOT + 1)
    )

    @functools.partial(
        pl.kernel,
        out_type=(
            jax.ShapeDtypeStruct((NPAD, F), jnp.float32),
            jax.ShapeDtypeStruct((NPAD, F), jnp.float32),
        ),
        mesh=_mesh(),
        scratch_types=scratch,
        name=name,
    )
    def k(sd_hbm, zeros_hbm, u_hbm, *rest):
        out0, out1 = rest[0], rest[1]
        utab, acc = rest[2], rest[3]
        sds = rest[4:4 + NSLOT]
        gbs = rest[4 + NSLOT:4 + 2 * NSLOT]
        sems = rest[4 + 2 * NSLOT:]
        semi = sems[:NSLOT]
        semg = sems[NSLOT:2 * NSLOT]
        semsc = sems[2 * NSLOT]

        c = lax.axis_index("c")
        s = lax.axis_index("s")
        w = c * NS + s

        r0 = s * RPT
        rows = pl.ds(r0, RPT)
        pltpu.async_copy(u_hbm.at[rows], utab.at[rows], sems[0])

        @pl.when(c == 0)
        def _():
            pltpu.async_copy(u_hbm.at[rows], acc.at[rows], sems[1])

        @pl.when(c == 1)
        def _():
            pltpu.async_copy(zeros_hbm.at[rows], acc.at[rows], sems[1])

        pltpu.make_async_copy(u_hbm.at[rows], utab.at[rows], sems[0]).wait()
        pltpu.make_async_copy(u_hbm.at[rows], acc.at[rows], sems[1]).wait()
        plsc.subcore_barrier()

        def idx_issue(j, b):
            pltpu.async_copy(sd_hbm.at[w, j], sds[b], semi[b])

        def idx_wait(j, b):
            pltpu.make_async_copy(sd_hbm.at[w, j], sds[b], semi[b]).wait()

        def gat_issue(b):
            pltpu.async_copy(utab.at[sds[b].at[0]], gbs[b], semg[b])

        def gat_wait(b):
            pltpu.make_async_copy(utab.at[sds[b].at[0]], gbs[b], semg[b]).wait()

        def sc_issue(b):
            pltpu.async_copy(gbs[b], acc.at[sds[b].at[1]], semsc, add=True)

        def sc_wait(b):
            pltpu.make_async_copy(gbs[b], acc.at[sds[b].at[1]], semsc).wait()

        idx_issue(0, 0)
        idx_issue(1, 1)
        idx_issue(2, 2)
        idx_wait(0, 0)
        gat_issue(0)
        idx_wait(1, 1)
        gat_issue(1)

        def step(jj, _):
            for b in range(NSLOT):
                j = jj * NSLOT + b
                gat_wait(b)

                @pl.when(j >= 1)
                def _(b4=(b + NSLOT - 1) % NSLOT):
                    sc_wait(b4)

                sc_issue(b)

                @pl.when(j + 3 < NCHUNK)
                def _(j=j, b3=(b + 3) % NSLOT):
                    idx_issue(j + 3, b3)

                @pl.when(j + 2 < NCHUNK)
                def _(j=j, b1=(b + 2) % NSLOT):
                    idx_wait(j + 2, b1)
                    gat_issue(b1)

            return 0

        lax.fori_loop(0, NCHUNK // NSLOT, step, 0)
        sc_wait((NCHUNK - 1) % NSLOT)
        plsc.subcore_barrier()

        r0 = s * RPT

        @pl.when(c == 0)
        def _():
            pltpu.sync_copy(acc.at[pl.ds(r0, RPT)], out0.at[pl.ds(r0, RPT)])

        @pl.when(c == 1)
        def _():
            pltpu.sync_copy(acc.at[pl.ds(r0, RPT)], out1.at[pl.ds(r0, RPT)])

    return k


_prop_id = _sc_prop("sc_prop")


@functools.partial(
    pl.kernel,
    out_type=jax.ShapeDtypeStruct((NC, NPAD, DW), jnp.float32),
    mesh=_mesh(),
    scratch_types=[
        pltpu.VMEM_SHARED((NPAD, DW), jnp.float32),
        pltpu.VMEM((1, CHD), jnp.int32),
        pltpu.VMEM((1, CHD), jnp.int32),
        pltpu.VMEM((CHD, DW), jnp.float32),
        pltpu.VMEM((RPT, DW), jnp.float32),
        pltpu.SemaphoreType.DMA,
        pltpu.SemaphoreType.DMA,
    ],
    name="sc_degree",
)
def _sc_degree(d_hbm, out, dacc, di0, di1, ones, zbuf, semi0, semi1):
    c = lax.axis_index("c")
    s = lax.axis_index("s")
    w = c * NS + s

    one = jnp.full((16,), 1.0, jnp.float32)
    zero = jnp.zeros((16,), jnp.float32)

    def fill_ones(i, _):
        ones[i, pl.ds(0, 16)] = one
        return 0

    lax.fori_loop(0, CHD, fill_ones, 0)

    def fill_zero(i, _):
        zbuf[i, pl.ds(0, 16)] = zero
        return 0

    lax.fori_loop(0, RPT, fill_zero, 0)
    pltpu.sync_copy(zbuf, dacc.at[pl.ds(s * RPT, RPT)])
    plsc.subcore_barrier()

    pltpu.async_copy(d_hbm.at[w, 0], di0.at[0], semi0)
    pltpu.async_copy(d_hbm.at[w, 1], di1.at[0], semi1)

    def step(jj, _):
        j0 = jj * 2
        j1 = j0 + 1
        pltpu.make_async_copy(d_hbm.at[w, j0], di0.at[0], semi0).wait()
        pltpu.sync_copy(ones, dacc.at[di0.at[0]], add=True)

        @pl.when(j0 + 2 < NCHD)
        def _():
            pltpu.async_copy(d_hbm.at[w, j0 + 2], di0.at[0], semi0)

        pltpu.make_async_copy(d_hbm.at[w, j1], di1.at[0], semi1).wait()
        pltpu.sync_copy(ones, dacc.at[di1.at[0]], add=True)

        @pl.when(j1 + 2 < NCHD)
        def _():
            pltpu.async_copy(d_hbm.at[w, j1 + 2], di1.at[0], semi1)

        return 0

    lax.fori_loop(0, NCHD // 2, step, 0)
    plsc.subcore_barrier()

    r0 = s * RPT
    pltpu.sync_copy(dacc.at[pl.ds(r0, RPT)], out.at[c, pl.ds(r0, RPT)])


@functools.partial(
    pl.kernel,
    out_type=jax.ShapeDtypeStruct((NW, NCHUNK, CH), jnp.float32),
    mesh=_mesh(),
    scratch_types=[
        pltpu.VMEM_SHARED((NPAD, F), jnp.float32),
    ]
    + [pltpu.VMEM((2, CH), jnp.int32) for _ in range(4)]
    + [pltpu.VMEM((CH, F), jnp.float32) for _ in range(4)]
    + [pltpu.VMEM((CH, F), jnp.float32) for _ in range(4)]
    + [
        pltpu.VMEM((CH,), jnp.float32),
    ]
    + [pltpu.SemaphoreType.DMA] * 12,
    name="sc_decode",
)
def _sc_decode(sd_hbm, z_hbm, out, ztab, *rest):
    sds = rest[0:4]
    gss = rest[4:8]
    gds = rest[8:12]
    pbuf = rest[12]
    semi = rest[13:17]
    sema = rest[17:21]
    semb = rest[21:25]
    c = lax.axis_index("c")
    s = lax.axis_index("s")
    w = c * NS + s

    r0 = s * RPT
    pltpu.sync_copy(z_hbm.at[pl.ds(r0, RPT)], ztab.at[pl.ds(r0, RPT)])
    plsc.subcore_barrier()

    def lane_perm(v, idx):
        dnums = lax.GatherDimensionNumbers(
            offset_dims=(), collapsed_slice_dims=(0,), start_index_map=(0,))
        return lax.gather(v, idx[:, None], dnums, (1,),
                          mode=lax.GatherScatterMode.PROMISE_IN_BOUNDS)

    def compute_chunk(j, gs, gd):
        lanes = lax.iota(jnp.int32, 16)

        def group(q, _):
            parts = []
            for e16 in range(16):
                e = q * 16 + e16
                t0 = gs[e, pl.ds(0, 16)] * gd[e, pl.ds(0, 16)]
                t1 = gs[e, pl.ds(16, 16)] * gd[e, pl.ds(16, 16)]
                t2 = gs[e, pl.ds(32, 16)] * gd[e, pl.ds(32, 16)]
                t3 = gs[e, pl.ds(48, 16)] * gd[e, pl.ds(48, 16)]
                d = (t0 + t1) + (t2 + t3)
                d = d + lane_perm(d, lanes ^ 8)
                d = d + lane_perm(d, lanes ^ 4)
                d = d + lane_perm(d, lanes ^ 2)
                d = d + lane_perm(d, lanes ^ 1)
                parts.append(jnp.where(lanes == e16, d, 0.0))
            while len(parts) > 1:
                parts = [a + b for a, b in zip(parts[::2], parts[1::2])]
            v = parts[0]
            pbuf[pl.ds(q * 16, 16)] = 1.0 / (1.0 + jnp.exp(-v))
            return 0

        lax.fori_loop(0, CH // 16, group, 0)
        pltpu.sync_copy(pbuf, out.at[w, j])

    def idx_issue(j, b):
        pltpu.async_copy(sd_hbm.at[w, j], sds[b], semi[b])

    def idx_wait(j, b):
        pltpu.make_async_copy(sd_hbm.at[w, j], sds[b], semi[b]).wait()

    def gat_issue(b):
        pltpu.async_copy(ztab.at[sds[b].at[0]], gss[b], sema[b])
        pltpu.async_copy(ztab.at[sds[b].at[1]], gds[b], semb[b])

    def gat_wait(b):
        pltpu.make_async_copy(ztab.at[sds[b].at[0]], gss[b], sema[b]).wait()
        pltpu.make_async_copy(ztab.at[sds[b].at[1]], gds[b], semb[b]).wait()

    idx_issue(0, 0)
    idx_issue(1, 1)
    idx_issue(2, 2)
    idx_wait(0, 0)
    gat_issue(0)
    idx_wait(1, 1)
    gat_issue(1)

    def step(jj, _):
        for b in range(4):
            j = jj * 4 + b
            gat_wait(b)

            @pl.when(j + 2 < NCHUNK)
            def _(j=j, b2=(b + 2) % 4):
                idx_wait(j + 2, b2)
                gat_issue(b2)

            @pl.when(j + 3 < NCHUNK)
            def _(j=j, b3=(b + 3) % 4):
                idx_issue(j + 3, b3)

            compute_chunk(j, gss[b], gds[b])
        return 0

    lax.fori_loop(0, NCHUNK // 4, step, 0)


BR = 1280
GRID = NPAD // BR


def _rowspec(width):
    return pl.BlockSpec((BR, width), lambda i: (i, 0))


def _fullspec(shape):
    nd = len(shape)
    return pl.BlockSpec(shape, lambda i, _nd=nd: (0,) * _nd)


def _tc_prologue(deg0, deg1, x_pad, w_proj):
    def body(d0, d1, x, wp, z0, u1, disb, d2b):
        deg = d0[:, :1] + d1[:, :1] + 1.0
        dis = lax.rsqrt(deg)
        z = lax.dot_general(x[...], wp[...], (((1,), (1,)), ((), ())),
                            precision=_HIGH, preferred_element_type=jnp.float32)
        z0[...] = z
        u1[...] = z * dis
        disb[...] = jnp.broadcast_to(dis, (BR, F))
        d2b[...] = jnp.broadcast_to(dis * dis, (BR, F))

    outs = [jax.ShapeDtypeStruct((NPAD, F), jnp.float32) for _ in range(4)]
    return pl.pallas_call(
        body,
        grid=(GRID,),
        in_specs=[_rowspec(DW), _rowspec(DW), _rowspec(IN_DIM),
                  _fullspec((F, IN_DIM))],
        out_specs=[_rowspec(F)] * 4,
        out_shape=outs,
    )(deg0, deg1, x_pad, w_proj)


def _tc_pre_bn(a0, a1, disb, wmat, bias, width):
    def finish(p, br, pre, stats):
        i = pl.program_id(0)

        @pl.when(i == 0)
        def _():
            stats[...] = jnp.zeros_like(stats)

        p = p + br[...]
        pre[...] = p
        rows = i * BR + lax.broadcasted_iota(jnp.int32, (BR, 1), 0)
        pm = jnp.where(rows < N, p, 0.0)
        stats[0:1, :] += jnp.sum(pm, axis=0, keepdims=True)
        stats[1:2, :] += jnp.sum(pm * pm, axis=0, keepdims=True)

    if wmat is not None:
        def fn(a0r, a1r, dr, wr, br, pre, stats):
            p0 = dr[...] * (a0r[...] + a1r[...])
            p = lax.dot_general(p0, wr[...], (((1,), (1,)), ((), ())),
                                precision=_HIGH,
                                preferred_element_type=jnp.float32)
            finish(p, br, pre, stats)

        in_specs = [_rowspec(F), _rowspec(F), _rowspec(F),
                    _fullspec((width, F)), _fullspec((1, width))]
        args = [a0, a1, disb, wmat, bias]
    else:
        def fn(a0r, a1r, dr, br, pre, stats):
            finish(dr[...] * (a0r[...] + a1r[...]), br, pre, stats)

        in_specs = [_rowspec(F), _rowspec(F), _rowspec(F),
                    _fullspec((1, width))]
        args = [a0, a1, disb, bias]
    return pl.pallas_call(
        fn,
        grid=(GRID,),
        in_specs=in_specs,
        out_specs=[_rowspec(width), _fullspec((2, width))],
        out_shape=[jax.ShapeDtypeStruct((NPAD, width), jnp.float32),
                   jax.ShapeDtypeStruct((2, width), jnp.float32)],
    )(*args)


def _tc_bn_relu_proj(pre, stats, gamma, beta, wmat, disb, width):
    def body(pr, st, gr, br, wr, dr, u):
        m = st[0:1, :] / float(N)
        v = st[1:2, :] / float(N) - m * m
        h = (pr[...] - m) * lax.rsqrt(v + EPS) * gr[...] + br[...]
        h = jnp.maximum(h, 0.0)
        mm = lax.dot_general(h, wr[...], (((1,), (1,)), ((), ())),
                             precision=_HIGH, preferred_element_type=jnp.float32)
        u[...] = dr[...] * mm

    return pl.pallas_call(
        body,
        grid=(GRID,),
        in_specs=[_rowspec(width), _fullspec((2, width)), _fullspec((1, width)),
                  _fullspec((1, width)), _fullspec((F, width)), _rowspec(F)],
        out_specs=_rowspec(F),
        out_shape=jax.ShapeDtypeStruct((NPAD, F), jnp.float32),
    )(pre, stats, gamma, beta, wmat, disb)


def _tc_bn_relu_v0(pre, stats, gamma, beta, disb):
    def body(pr, st, gr, br, dr, v0, g):
        m = st[0:1, :] / float(N)
        v = st[1:2, :] / float(N) - m * m
        h = (pr[...] - m) * lax.rsqrt(v + EPS) * gr[...] + br[...]
        h = jnp.maximum(h, 0.0)
        vv = dr[...] * h
        v0[...] = vv
        g[...] = ALPHA * vv

    return pl.pallas_call(
        body,
        grid=(GRID,),
        in_specs=[_rowspec(F), _fullspec((2, F)), _fullspec((1, F)),
                  _fullspec((1, F)), _rowspec(F)],
        out_specs=[_rowspec(F)] * 2,
        out_shape=[jax.ShapeDtypeStruct((NPAD, F), jnp.float32)] * 2,
    )(pre, stats, gamma, beta, disb)


def _tc_appnp_stage(a0, a1, d2b, g):
    def body(a0r, a1r, d2r, gr, u):
        u[...] = (1.0 - ALPHA) * d2r[...] * (a0r[...] + a1r[...]) + gr[...]

    return pl.pallas_call(
        body,
        grid=(GRID,),
        in_specs=[_rowspec(F)] * 4,
        out_specs=_rowspec(F),
        out_shape=jax.ShapeDtypeStruct((NPAD, F), jnp.float32),
    )(a0, a1, d2b, g)


def _tc_finish(a0, a1, d2b, g, disb, z0):
    def body(a0r, a1r, d2r, gr, dr, z0r, zf):
        v = (1.0 - ALPHA) * d2r[...] * (a0r[...] + a1r[...]) + gr[...]
        zf[...] = v / dr[...] + z0r[...]

    return pl.pallas_call(
        body,
        grid=(GRID,),
        in_specs=[_rowspec(F)] * 6,
        out_specs=_rowspec(F),
        out_shape=jax.ShapeDtypeStruct((NPAD, F), jnp.float32),
    )(a0, a1, d2b, g, disb, z0)


def kernel(x, edge_index, W_proj, W1, b1, gamma1, beta1, W2, b2, gamma2, beta2):
    src = edge_index[0]
    dst = edge_index[1]

    pad_n = SLOTS - EPT
    w_ids = jnp.arange(NW, dtype=jnp.int32)[:, None]
    k_ids = jnp.arange(pad_n, dtype=jnp.int32)[None, :]
    pad_rows = N + (w_ids * 7 + k_ids) % (NPAD - N)
    srcp = jnp.concatenate([src.reshape(NW, EPT), pad_rows], axis=1)
    dstp = jnp.concatenate([dst.reshape(NW, EPT), pad_rows], axis=1)
    sdp = jnp.stack([srcp.reshape(NW, NCHUNK, CH),
                     dstp.reshape(NW, NCHUNK, CH)], axis=2)
    dstp_deg = dstp.reshape(NW, NCHD, CHD)

    x_pad = jnp.pad(x, ((0, NPAD - N), (0, 0)))
    znode = jnp.zeros((NPAD, F), jnp.float32)

    degs = _sc_degree(dstp_deg)
    z0, u1, disb, d2b = _tc_prologue(degs[0], degs[1], x_pad, W_proj)

    a0, a1 = _prop_id(sdp, znode, u1)
    pre1, stats1 = _tc_pre_bn(a0, a1, disb, W1, b1.reshape(1, HID), HID)
    u2 = _tc_bn_relu_proj(pre1, stats1, gamma1.reshape(1, HID),
                          beta1.reshape(1, HID), W2, disb, HID)

    a0, a1 = _prop_id(sdp, znode, u2)
    pre2, stats2 = _tc_pre_bn(a0, a1, disb, None, b2.reshape(1, F), F)
    v0, g = _tc_bn_relu_v0(pre2, stats2, gamma2.reshape(1, F),
                           beta2.reshape(1, F), disb)

    a0, a1 = _prop_id(sdp, znode, v0)
    for _ in range(K_PROP - 1):
        u = _tc_appnp_stage(a0, a1, d2b, g)
        a0, a1 = _prop_id(sdp, znode, u)

    z_final_pad = _tc_finish(a0, a1, d2b, g, disb, z0)

    dec = _sc_decode(sdp, z_final_pad)
    probs = dec.reshape(NW, SLOTS)[:, :EPT].reshape(E)
    return probs, z_final_pad[:N]


# speedup vs baseline: 1.0150x; 1.0150x over previous
import functools

import jax
import jax.numpy as jnp
from jax import lax
from jax.experimental import pallas as pl
from jax.experimental.pallas import tpu as pltpu
from jax.experimental.pallas import tpu_sc as plsc

N = 10000
E = 320000
IN_DIM = 128
HID = 128
F = 64
K_PROP = 10
ALPHA = 0.1
EPS = 1e-5

NC, NS = 2, 16
NW = NC * NS
NPAD = 10240
RPT = NPAD // NS
RB = 64
NBLK = RPT // RB
EPT = E // NW
CH = 80
NCHUNK = 128
NSLOT = 4
SLOTS = NCHUNK * CH
DW = 16
CHD = 128
NCHD = SLOTS // CHD

_HIGH = lax.Precision.HIGHEST


def _mesh():
    return plsc.VectorSubcoreMesh(
        core_axis_name="c", subcore_axis_name="s", num_cores=NC, num_subcores=NS
    )


def _sc_prop(name):
    scratch = (
        [
            pltpu.VMEM_SHARED((NPAD, F), jnp.float32),
            pltpu.VMEM_SHARED((NPAD, F), jnp.float32),
        ]
        + [pltpu.VMEM((2, CH), jnp.int32) for _ in range(NSLOT)]
        + [pltpu.VMEM((CH, F), jnp.float32) for _ in range(NSLOT)]
        + [pltpu.SemaphoreType.DMA] * (2 * NSLOT + 1)
    )

    @functools.partial(
        pl.kernel,
        out_type=(
            jax.ShapeDtypeStruct((NPAD, F), jnp.float32),
            jax.ShapeDtypeStruct((NPAD, F), jnp.float32),
        ),
        mesh=_mesh(),
        scratch_types=scratch,
        name=name,
    )
    def k(sd_hbm, zeros_hbm, u_hbm, *rest):
        out0, out1 = rest[0], rest[1]
        utab, acc = rest[2], rest[3]
        sds = rest[4:4 + NSLOT]
        gbs = rest[4 + NSLOT:4 + 2 * NSLOT]
        sems = rest[4 + 2 * NSLOT:]
        semi = sems[:NSLOT]
        semg = sems[NSLOT:2 * NSLOT]
        semsc = sems[2 * NSLOT]

        c = lax.axis_index("c")
        s = lax.axis_index("s")
        w = c * NS + s

        r0 = s * RPT
        pltpu.sync_copy(u_hbm.at[pl.ds(r0, RPT)], utab.at[pl.ds(r0, RPT)])

        @pl.when(c == 0)
        def _():
            pltpu.sync_copy(u_hbm.at[pl.ds(r0, RPT)], acc.at[pl.ds(r0, RPT)])

        @pl.when(c == 1)
        def _():
            pltpu.sync_copy(zeros_hbm.at[pl.ds(r0, RPT)],
                            acc.at[pl.ds(r0, RPT)])

        plsc.subcore_barrier()

        def idx_issue(j, b):
            pltpu.async_copy(sd_hbm.at[w, j], sds[b], semi[b])

        def idx_wait(j, b):
            pltpu.make_async_copy(sd_hbm.at[w, j], sds[b], semi[b]).wait()

        def gat_issue(b):
            pltpu.async_copy(utab.at[sds[b].at[0]], gbs[b], semg[b])

        def gat_wait(b):
            pltpu.make_async_copy(utab.at[sds[b].at[0]], gbs[b], semg[b]).wait()

        def sc_issue(b):
            pltpu.async_copy(gbs[b], acc.at[sds[b].at[1]], semsc, add=True)

        def sc_wait(b):
            pltpu.make_async_copy(gbs[b], acc.at[sds[b].at[1]], semsc).wait()

        idx_issue(0, 0)
        idx_issue(1, 1)
        idx_issue(2, 2)
        idx_wait(0, 0)
        gat_issue(0)
        idx_wait(1, 1)
        gat_issue(1)

        def step(jj, _):
            for b in range(NSLOT):
                j = jj * NSLOT + b
                gat_wait(b)

                @pl.when(j >= 1)
                def _(b4=(b + NSLOT - 1) % NSLOT):
                    sc_wait(b4)

                sc_issue(b)

                @pl.when(j + 3 < NCHUNK)
                def _(j=j, b3=(b + 3) % NSLOT):
                    idx_issue(j + 3, b3)

                @pl.when(j + 2 < NCHUNK)
                def _(j=j, b1=(b + 2) % NSLOT):
                    idx_wait(j + 2, b1)
                    gat_issue(b1)

            return 0

        lax.fori_loop(0, NCHUNK // NSLOT, step, 0)
        sc_wait((NCHUNK - 1) % NSLOT)
        plsc.subcore_barrier()

        r0 = s * RPT

        @pl.when(c == 0)
        def _():
            pltpu.sync_copy(acc.at[pl.ds(r0, RPT)], out0.at[pl.ds(r0, RPT)])

        @pl.when(c == 1)
        def _():
            pltpu.sync_copy(acc.at[pl.ds(r0, RPT)], out1.at[pl.ds(r0, RPT)])

    return k


_prop_id = _sc_prop("sc_prop")


@functools.partial(
    pl.kernel,
    out_type=jax.ShapeDtypeStruct((NC, NPAD, DW), jnp.float32),
    mesh=_mesh(),
    scratch_types=[
        pltpu.VMEM_SHARED((NPAD, DW), jnp.float32),
        pltpu.VMEM((1, CHD), jnp.int32),
        pltpu.VMEM((1, CHD), jnp.int32),
        pltpu.VMEM((CHD, DW), jnp.float32),
        pltpu.VMEM((RPT, DW), jnp.float32),
        pltpu.SemaphoreType.DMA,
        pltpu.SemaphoreType.DMA,
    ],
    name="sc_degree",
)
def _sc_degree(d_hbm, out, dacc, di0, di1, ones, zbuf, semi0, semi1):
    c = lax.axis_index("c")
    s = lax.axis_index("s")
    w = c * NS + s

    one = jnp.full((16,), 1.0, jnp.float32)
    zero = jnp.zeros((16,), jnp.float32)

    def fill_ones(i, _):
        ones[i, pl.ds(0, 16)] = one
        return 0

    lax.fori_loop(0, CHD, fill_ones, 0)

    def fill_zero(i, _):
        zbuf[i, pl.ds(0, 16)] = zero
        return 0

    lax.fori_loop(0, RPT, fill_zero, 0)
    pltpu.sync_copy(zbuf, dacc.at[pl.ds(s * RPT, RPT)])
    plsc.subcore_barrier()

    pltpu.async_copy(d_hbm.at[w, 0], di0.at[0], semi0)
    pltpu.async_copy(d_hbm.at[w, 1], di1.at[0], semi1)

    def step(jj, _):
        j0 = jj * 2
        j1 = j0 + 1
        pltpu.make_async_copy(d_hbm.at[w, j0], di0.at[0], semi0).wait()
        pltpu.sync_copy(ones, dacc.at[di0.at[0]], add=True)

        @pl.when(j0 + 2 < NCHD)
        def _():
            pltpu.async_copy(d_hbm.at[w, j0 + 2], di0.at[0], semi0)

        pltpu.make_async_copy(d_hbm.at[w, j1], di1.at[0], semi1).wait()
        pltpu.sync_copy(ones, dacc.at[di1.at[0]], add=True)

        @pl.when(j1 + 2 < NCHD)
        def _():
            pltpu.async_copy(d_hbm.at[w, j1 + 2], di1.at[0], semi1)

        return 0

    lax.fori_loop(0, NCHD // 2, step, 0)
    plsc.subcore_barrier()

    r0 = s * RPT
    pltpu.sync_copy(dacc.at[pl.ds(r0, RPT)], out.at[c, pl.ds(r0, RPT)])


@functools.partial(
    pl.kernel,
    out_type=jax.ShapeDtypeStruct((NW, NCHUNK, CH), jnp.float32),
    mesh=_mesh(),
    scratch_types=[
        pltpu.VMEM_SHARED((NPAD, F), jnp.float32),
        pltpu.VMEM((2, CH), jnp.int32),
        pltpu.VMEM((2, CH), jnp.int32),
        pltpu.VMEM((CH, F), jnp.float32),
        pltpu.VMEM((CH, F), jnp.float32),
        pltpu.VMEM((CH, F), jnp.float32),
        pltpu.VMEM((CH, F), jnp.float32),
        pltpu.VMEM((CH,), jnp.float32),
        pltpu.SemaphoreType.DMA,
        pltpu.SemaphoreType.DMA,
        pltpu.SemaphoreType.DMA,
        pltpu.SemaphoreType.DMA,
        pltpu.SemaphoreType.DMA,
        pltpu.SemaphoreType.DMA,
    ],
    name="sc_decode",
)
def _sc_decode(sd_hbm, z_hbm, out,
               ztab, sd0, sd1, gs0, gd0, gs1, gd1, pbuf,
               si0, si1, sa0, sb0, sa1, sb1):
    c = lax.axis_index("c")
    s = lax.axis_index("s")
    w = c * NS + s

    r0 = s * RPT
    pltpu.sync_copy(z_hbm.at[pl.ds(r0, RPT)], ztab.at[pl.ds(r0, RPT)])
    plsc.subcore_barrier()

    def lane_perm(v, idx):
        dnums = lax.GatherDimensionNumbers(
            offset_dims=(), collapsed_slice_dims=(0,), start_index_map=(0,))
        return lax.gather(v, idx[:, None], dnums, (1,),
                          mode=lax.GatherScatterMode.PROMISE_IN_BOUNDS)

    def compute_chunk(j, gs, gd):
        lanes = lax.iota(jnp.int32, 16)

        def group(q, _):
            parts = []
            for e16 in range(16):
                e = q * 16 + e16
                t0 = gs[e, pl.ds(0, 16)] * gd[e, pl.ds(0, 16)]
                t1 = gs[e, pl.ds(16, 16)] * gd[e, pl.ds(16, 16)]
                t2 = gs[e, pl.ds(32, 16)] * gd[e, pl.ds(32, 16)]
                t3 = gs[e, pl.ds(48, 16)] * gd[e, pl.ds(48, 16)]
                d = (t0 + t1) + (t2 + t3)
                d = d + lane_perm(d, lanes ^ 8)
                d = d + lane_perm(d, lanes ^ 4)
                d = d + lane_perm(d, lanes ^ 2)
                d = d + lane_perm(d, lanes ^ 1)
                parts.append(jnp.where(lanes == e16, d, 0.0))
            while len(parts) > 1:
                parts = [a + b for a, b in zip(parts[::2], parts[1::2])]
            v = parts[0]
            pbuf[pl.ds(q * 16, 16)] = 1.0 / (1.0 + jnp.exp(-v))
            return 0

        lax.fori_loop(0, CH // 16, group, 0)
        pltpu.sync_copy(pbuf, out.at[w, j])

    pltpu.async_copy(sd_hbm.at[w, 0], sd0, si0)
    pltpu.async_copy(sd_hbm.at[w, 1], sd1, si1)

    def step(jj, _):
        j0 = jj * 2
        j1 = j0 + 1
        pltpu.make_async_copy(sd_hbm.at[w, j0], sd0, si0).wait()
        pltpu.async_copy(ztab.at[sd0.at[0]], gs0, sa0)
        pltpu.async_copy(ztab.at[sd0.at[1]], gd0, sb0)
        pltpu.make_async_copy(sd_hbm.at[w, j1], sd1, si1).wait()
        pltpu.async_copy(ztab.at[sd1.at[0]], gs1, sa1)
        pltpu.async_copy(ztab.at[sd1.at[1]], gd1, sb1)

        pltpu.make_async_copy(ztab.at[sd0.at[0]], gs0, sa0).wait()
        pltpu.make_async_copy(ztab.at[sd0.at[1]], gd0, sb0).wait()
        compute_chunk(j0, gs0, gd0)

        @pl.when(j0 + 2 < NCHUNK)
        def _():
            pltpu.async_copy(sd_hbm.at[w, j0 + 2], sd0, si0)

        pltpu.make_async_copy(ztab.at[sd1.at[0]], gs1, sa1).wait()
        pltpu.make_async_copy(ztab.at[sd1.at[1]], gd1, sb1).wait()
        compute_chunk(j1, gs1, gd1)

        @pl.when(j1 + 2 < NCHUNK)
        def _():
            pltpu.async_copy(sd_hbm.at[w, j1 + 2], sd1, si1)

        return 0

    lax.fori_loop(0, NCHUNK // 2, step, 0)


BR = 1280
GRID = NPAD // BR


def _rowspec(width):
    return pl.BlockSpec((BR, width), lambda i: (i, 0))


def _fullspec(shape):
    nd = len(shape)
    return pl.BlockSpec(shape, lambda i, _nd=nd: (0,) * _nd)


def _tc_prologue(deg0, deg1, x_pad, w_proj):
    def body(d0, d1, x, wp, z0, u1, disb, d2b):
        deg = d0[:, :1] + d1[:, :1] + 1.0
        dis = lax.rsqrt(deg)
        z = lax.dot_general(x[...], wp[...], (((1,), (1,)), ((), ())),
                            precision=_HIGH, preferred_element_type=jnp.float32)
        z0[...] = z
        u1[...] = z * dis
        disb[...] = jnp.broadcast_to(dis, (BR, F))
        d2b[...] = jnp.broadcast_to(dis * dis, (BR, F))

    outs = [jax.ShapeDtypeStruct((NPAD, F), jnp.float32) for _ in range(4)]
    return pl.pallas_call(
        body,
        grid=(GRID,),
        in_specs=[_rowspec(DW), _rowspec(DW), _rowspec(IN_DIM),
                  _fullspec((F, IN_DIM))],
        out_specs=[_rowspec(F)] * 4,
        out_shape=outs,
    )(deg0, deg1, x_pad, w_proj)


def _tc_pre_bn(a0, a1, disb, wmat, bias, width):
    def finish(p, br, pre, stats):
        i = pl.program_id(0)

        @pl.when(i == 0)
        def _():
            stats[...] = jnp.zeros_like(stats)

        p = p + br[...]
        pre[...] = p
        rows = i * BR + lax.broadcasted_iota(jnp.int32, (BR, 1), 0)
        pm = jnp.where(rows < N, p, 0.0)
        stats[0:1, :] += jnp.sum(pm, axis=0, keepdims=True)
        stats[1:2, :] += jnp.sum(pm * pm, axis=0, keepdims=True)

    if wmat is not None:
        def fn(a0r, a1r, dr, wr, br, pre, stats):
            p0 = dr[...] * (a0r[...] + a1r[...])
            p = lax.dot_general(p0, wr[...], (((1,), (1,)), ((), ())),
                                precision=_HIGH,
                                preferred_element_type=jnp.float32)
            finish(p, br, pre, stats)

        in_specs = [_rowspec(F), _rowspec(F), _rowspec(F),
                    _fullspec((width, F)), _fullspec((1, width))]
        args = [a0, a1, disb, wmat, bias]
    else:
        def fn(a0r, a1r, dr, br, pre, stats):
            finish(dr[...] * (a0r[...] + a1r[...]), br, pre, stats)

        in_specs = [_rowspec(F), _rowspec(F), _rowspec(F),
                    _fullspec((1, width))]
        args = [a0, a1, disb, bias]
    return pl.pallas_call(
        fn,
        grid=(GRID,),
        in_specs=in_specs,
        out_specs=[_rowspec(width), _fullspec((2, width))],
        out_shape=[jax.ShapeDtypeStruct((NPAD, width), jnp.float32),
                   jax.ShapeDtypeStruct((2, width), jnp.float32)],
    )(*args)


def _tc_bn_relu_proj(pre, stats, gamma, beta, wmat, disb, width):
    def body(pr, st, gr, br, wr, dr, u):
        m = st[0:1, :] / float(N)
        v = st[1:2, :] / float(N) - m * m
        h = (pr[...] - m) * lax.rsqrt(v + EPS) * gr[...] + br[...]
        h = jnp.maximum(h, 0.0)
        mm = lax.dot_general(h, wr[...], (((1,), (1,)), ((), ())),
                             precision=_HIGH, preferred_element_type=jnp.float32)
        u[...] = dr[...] * mm

    return pl.pallas_call(
        body,
        grid=(GRID,),
        in_specs=[_rowspec(width), _fullspec((2, width)), _fullspec((1, width)),
                  _fullspec((1, width)), _fullspec((F, width)), _rowspec(F)],
        out_specs=_rowspec(F),
        out_shape=jax.ShapeDtypeStruct((NPAD, F), jnp.float32),
    )(pre, stats, gamma, beta, wmat, disb)


def _tc_bn_relu_v0(pre, stats, gamma, beta, disb):
    def body(pr, st, gr, br, dr, v0, g):
        m = st[0:1, :] / float(N)
        v = st[1:2, :] / float(N) - m * m
        h = (pr[...] - m) * lax.rsqrt(v + EPS) * gr[...] + br[...]
        h = jnp.maximum(h, 0.0)
        vv = dr[...] * h
        v0[...] = vv
        g[...] = ALPHA * vv

    return pl.pallas_call(
        body,
        grid=(GRID,),
        in_specs=[_rowspec(F), _fullspec((2, F)), _fullspec((1, F)),
                  _fullspec((1, F)), _rowspec(F)],
        out_specs=[_rowspec(F)] * 2,
        out_shape=[jax.ShapeDtypeStruct((NPAD, F), jnp.float32)] * 2,
    )(pre, stats, gamma, beta, disb)


def _tc_appnp_stage(a0, a1, d2b, g):
    def body(a0r, a1r, d2r, gr, u):
        u[...] = (1.0 - ALPHA) * d2r[...] * (a0r[...] + a1r[...]) + gr[...]

    return pl.pallas_call(
        body,
        grid=(GRID,),
        in_specs=[_rowspec(F)] * 4,
        out_specs=_rowspec(F),
        out_shape=jax.ShapeDtypeStruct((NPAD, F), jnp.float32),
    )(a0, a1, d2b, g)


def _tc_finish(a0, a1, d2b, g, disb, z0):
    def body(a0r, a1r, d2r, gr, dr, z0r, zf):
        v = (1.0 - ALPHA) * d2r[...] * (a0r[...] + a1r[...]) + gr[...]
        zf[...] = v / dr[...] + z0r[...]

    return pl.pallas_call(
        body,
        grid=(GRID,),
        in_specs=[_rowspec(F)] * 6,
        out_specs=_rowspec(F),
        out_shape=jax.ShapeDtypeStruct((NPAD, F), jnp.float32),
    )(a0, a1, d2b, g, disb, z0)


def kernel(x, edge_index, W_proj, W1, b1, gamma1, beta1, W2, b2, gamma2, beta2):
    src = edge_index[0]
    dst = edge_index[1]

    pad_n = SLOTS - EPT
    w_ids = jnp.arange(NW, dtype=jnp.int32)[:, None]
    k_ids = jnp.arange(pad_n, dtype=jnp.int32)[None, :]
    pad_rows = N + (w_ids * 7 + k_ids) % (NPAD - N)
    srcp = jnp.concatenate([src.reshape(NW, EPT), pad_rows], axis=1)
    dstp = jnp.concatenate([dst.reshape(NW, EPT), pad_rows], axis=1)
    sdp = jnp.stack([srcp.reshape(NW, NCHUNK, CH),
                     dstp.reshape(NW, NCHUNK, CH)], axis=2)
    dstp_deg = dstp.reshape(NW, NCHD, CHD)

    x_pad = jnp.pad(x, ((0, NPAD - N), (0, 0)))
    znode = jnp.zeros((NPAD, F), jnp.float32)

    degs = _sc_degree(dstp_deg)
    z0, u1, disb, d2b = _tc_prologue(degs[0], degs[1], x_pad, W_proj)

    a0, a1 = _prop_id(sdp, znode, u1)
    pre1, stats1 = _tc_pre_bn(a0, a1, disb, W1, b1.reshape(1, HID), HID)
    u2 = _tc_bn_relu_proj(pre1, stats1, gamma1.reshape(1, HID),
                          beta1.reshape(1, HID), W2, disb, HID)

    a0, a1 = _prop_id(sdp, znode, u2)
    pre2, stats2 = _tc_pre_bn(a0, a1, disb, None, b2.reshape(1, F), F)
    v0, g = _tc_bn_relu_v0(pre2, stats2, gamma2.reshape(1, F),
                           beta2.reshape(1, F), disb)

    a0, a1 = _prop_id(sdp, znode, v0)
    for _ in range(K_PROP - 1):
        u = _tc_appnp_stage(a0, a1, d2b, g)
        a0, a1 = _prop_id(sdp, znode, u)

    z_final_pad = _tc_finish(a0, a1, d2b, g, disb, z0)

    dec = _sc_decode(sdp, z_final_pad)
    probs = dec.reshape(NW, SLOTS)[:, :EPT].reshape(E)
    return probs, z_final_pad[:N]


# speedup vs baseline: 1.0175x; 1.0025x over previous
import functools

import jax
import jax.numpy as jnp
from jax import lax
from jax.experimental import pallas as pl
from jax.experimental.pallas import tpu as pltpu
from jax.experimental.pallas import tpu_sc as plsc

N = 10000
E = 320000
IN_DIM = 128
HID = 128
F = 64
K_PROP = 10
ALPHA = 0.1
EPS = 1e-5

NC, NS = 2, 16
NW = NC * NS
NPAD = 10240
RPT = NPAD // NS
RB = 64
NBLK = RPT // RB
EPT = E // NW
CH = 64
NCHUNK = 160
NSLOT = 5
SLOTS = NCHUNK * CH
DW = 16
CHD = 128
NCHD = SLOTS // CHD

_HIGH = lax.Precision.HIGHEST


def _mesh():
    return plsc.VectorSubcoreMesh(
        core_axis_name="c", subcore_axis_name="s", num_cores=NC, num_subcores=NS
    )


def _sc_prop(name):
    scratch = (
        [
            pltpu.VMEM_SHARED((NPAD, F), jnp.float32),
            pltpu.VMEM_SHARED((NPAD, F), jnp.float32),
        ]
        + [pltpu.VMEM((2, CH), jnp.int32) for _ in range(NSLOT)]
        + [pltpu.VMEM((CH, F), jnp.float32) for _ in range(NSLOT)]
        + [pltpu.SemaphoreType.DMA] * (2 * NSLOT + 1)
    )

    @functools.partial(
        pl.kernel,
        out_type=(
            jax.ShapeDtypeStruct((NPAD, F), jnp.float32),
            jax.ShapeDtypeStruct((NPAD, F), jnp.float32),
        ),
        mesh=_mesh(),
        scratch_types=scratch,
        name=name,
    )
    def k(sd_hbm, zeros_hbm, u_hbm, *rest):
        out0, out1 = rest[0], rest[1]
        utab, acc = rest[2], rest[3]
        sds = rest[4:4 + NSLOT]
        gbs = rest[4 + NSLOT:4 + 2 * NSLOT]
        sems = rest[4 + 2 * NSLOT:]
        semi = sems[:NSLOT]
        semg = sems[NSLOT:2 * NSLOT]
        semsc = sems[2 * NSLOT]

        c = lax.axis_index("c")
        s = lax.axis_index("s")
        w = c * NS + s

        r0 = s * RPT
        pltpu.sync_copy(u_hbm.at[pl.ds(r0, RPT)], utab.at[pl.ds(r0, RPT)])

        @pl.when(c == 0)
        def _():
            pltpu.sync_copy(u_hbm.at[pl.ds(r0, RPT)], acc.at[pl.ds(r0, RPT)])

        @pl.when(c == 1)
        def _():
            pltpu.sync_copy(zeros_hbm.at[pl.ds(r0, RPT)],
                            acc.at[pl.ds(r0, RPT)])

        plsc.subcore_barrier()

        def idx_issue(j, b):
            pltpu.async_copy(sd_hbm.at[w, j], sds[b], semi[b])

        def idx_wait(j, b):
            pltpu.make_async_copy(sd_hbm.at[w, j], sds[b], semi[b]).wait()

        def gat_issue(b):
            pltpu.async_copy(utab.at[sds[b].at[0]], gbs[b], semg[b])

        def gat_wait(b):
            pltpu.make_async_copy(utab.at[sds[b].at[0]], gbs[b], semg[b]).wait()

        def sc_issue(b):
            pltpu.async_copy(gbs[b], acc.at[sds[b].at[1]], semsc, add=True)

        def sc_wait(b):
            pltpu.make_async_copy(gbs[b], acc.at[sds[b].at[1]], semsc).wait()

        idx_issue(0, 0)
        idx_issue(1, 1)
        idx_issue(2, 2)
        idx_wait(0, 0)
        gat_issue(0)
        idx_wait(1, 1)
        gat_issue(1)

        def step(jj, _):
            for b in range(NSLOT):
                j = jj * NSLOT + b
                gat_wait(b)

                @pl.when(j >= 1)
                def _(b4=(b + NSLOT - 1) % NSLOT):
                    sc_wait(b4)

                sc_issue(b)

                @pl.when(j + 3 < NCHUNK)
                def _(j=j, b3=(b + 3) % NSLOT):
                    idx_issue(j + 3, b3)

                @pl.when(j + 2 < NCHUNK)
                def _(j=j, b1=(b + 2) % NSLOT):
                    idx_wait(j + 2, b1)
                    gat_issue(b1)

            return 0

        lax.fori_loop(0, NCHUNK // NSLOT, step, 0)
        sc_wait((NCHUNK - 1) % NSLOT)
        plsc.subcore_barrier()

        r0 = s * RPT

        @pl.when(c == 0)
        def _():
            pltpu.sync_copy(acc.at[pl.ds(r0, RPT)], out0.at[pl.ds(r0, RPT)])

        @pl.when(c == 1)
        def _():
            pltpu.sync_copy(acc.at[pl.ds(r0, RPT)], out1.at[pl.ds(r0, RPT)])

    return k


_prop_id = _sc_prop("sc_prop")


@functools.partial(
    pl.kernel,
    out_type=jax.ShapeDtypeStruct((NC, NPAD, DW), jnp.float32),
    mesh=_mesh(),
    scratch_types=[
        pltpu.VMEM_SHARED((NPAD, DW), jnp.float32),
        pltpu.VMEM((1, CHD), jnp.int32),
        pltpu.VMEM((1, CHD), jnp.int32),
        pltpu.VMEM((CHD, DW), jnp.float32),
        pltpu.VMEM((RPT, DW), jnp.float32),
        pltpu.SemaphoreType.DMA,
        pltpu.SemaphoreType.DMA,
    ],
    name="sc_degree",
)
def _sc_degree(d_hbm, out, dacc, di0, di1, ones, zbuf, semi0, semi1):
    c = lax.axis_index("c")
    s = lax.axis_index("s")
    w = c * NS + s

    one = jnp.full((16,), 1.0, jnp.float32)
    zero = jnp.zeros((16,), jnp.float32)

    def fill_ones(i, _):
        ones[i, pl.ds(0, 16)] = one
        return 0

    lax.fori_loop(0, CHD, fill_ones, 0)

    def fill_zero(i, _):
        zbuf[i, pl.ds(0, 16)] = zero
        return 0

    lax.fori_loop(0, RPT, fill_zero, 0)
    pltpu.sync_copy(zbuf, dacc.at[pl.ds(s * RPT, RPT)])
    plsc.subcore_barrier()

    pltpu.async_copy(d_hbm.at[w, 0], di0.at[0], semi0)
    pltpu.async_copy(d_hbm.at[w, 1], di1.at[0], semi1)

    def step(jj, _):
        j0 = jj * 2
        j1 = j0 + 1
        pltpu.make_async_copy(d_hbm.at[w, j0], di0.at[0], semi0).wait()
        pltpu.sync_copy(ones, dacc.at[di0.at[0]], add=True)

        @pl.when(j0 + 2 < NCHD)
        def _():
            pltpu.async_copy(d_hbm.at[w, j0 + 2], di0.at[0], semi0)

        pltpu.make_async_copy(d_hbm.at[w, j1], di1.at[0], semi1).wait()
        pltpu.sync_copy(ones, dacc.at[di1.at[0]], add=True)

        @pl.when(j1 + 2 < NCHD)
        def _():
            pltpu.async_copy(d_hbm.at[w, j1 + 2], di1.at[0], semi1)

        return 0

    lax.fori_loop(0, NCHD // 2, step, 0)
    plsc.subcore_barrier()

    r0 = s * RPT
    pltpu.sync_copy(dacc.at[pl.ds(r0, RPT)], out.at[c, pl.ds(r0, RPT)])


@functools.partial(
    pl.kernel,
    out_type=jax.ShapeDtypeStruct((NW, NCHUNK, CH), jnp.float32),
    mesh=_mesh(),
    scratch_types=[
        pltpu.VMEM_SHARED((NPAD, F), jnp.float32),
        pltpu.VMEM((2, CH), jnp.int32),
        pltpu.VMEM((2, CH), jnp.int32),
        pltpu.VMEM((CH, F), jnp.float32),
        pltpu.VMEM((CH, F), jnp.float32),
        pltpu.VMEM((CH, F), jnp.float32),
        pltpu.VMEM((CH, F), jnp.float32),
        pltpu.VMEM((CH,), jnp.float32),
        pltpu.SemaphoreType.DMA,
        pltpu.SemaphoreType.DMA,
        pltpu.SemaphoreType.DMA,
        pltpu.SemaphoreType.DMA,
        pltpu.SemaphoreType.DMA,
        pltpu.SemaphoreType.DMA,
    ],
    name="sc_decode",
)
def _sc_decode(sd_hbm, z_hbm, out,
               ztab, sd0, sd1, gs0, gd0, gs1, gd1, pbuf,
               si0, si1, sa0, sb0, sa1, sb1):
    c = lax.axis_index("c")
    s = lax.axis_index("s")
    w = c * NS + s

    r0 = s * RPT
    pltpu.sync_copy(z_hbm.at[pl.ds(r0, RPT)], ztab.at[pl.ds(r0, RPT)])
    plsc.subcore_barrier()

    def lane_perm(v, idx):
        dnums = lax.GatherDimensionNumbers(
            offset_dims=(), collapsed_slice_dims=(0,), start_index_map=(0,))
        return lax.gather(v, idx[:, None], dnums, (1,),
                          mode=lax.GatherScatterMode.PROMISE_IN_BOUNDS)

    def compute_chunk(j, gs, gd):
        lanes = lax.iota(jnp.int32, 16)

        def group(q, _):
            parts = []
            for e16 in range(16):
                e = q * 16 + e16
                t0 = gs[e, pl.ds(0, 16)] * gd[e, pl.ds(0, 16)]
                t1 = gs[e, pl.ds(16, 16)] * gd[e, pl.ds(16, 16)]
                t2 = gs[e, pl.ds(32, 16)] * gd[e, pl.ds(32, 16)]
                t3 = gs[e, pl.ds(48, 16)] * gd[e, pl.ds(48, 16)]
                d = (t0 + t1) + (t2 + t3)
                d = d + lane_perm(d, lanes ^ 8)
                d = d + lane_perm(d, lanes ^ 4)
                d = d + lane_perm(d, lanes ^ 2)
                d = d + lane_perm(d, lanes ^ 1)
                parts.append(jnp.where(lanes == e16, d, 0.0))
            while len(parts) > 1:
                parts = [a + b for a, b in zip(parts[::2], parts[1::2])]
            v = parts[0]
            pbuf[pl.ds(q * 16, 16)] = 1.0 / (1.0 + jnp.exp(-v))
            return 0

        lax.fori_loop(0, CH // 16, group, 0)
        pltpu.sync_copy(pbuf, out.at[w, j])

    pltpu.async_copy(sd_hbm.at[w, 0], sd0, si0)
    pltpu.async_copy(sd_hbm.at[w, 1], sd1, si1)

    def step(jj, _):
        j0 = jj * 2
        j1 = j0 + 1
        pltpu.make_async_copy(sd_hbm.at[w, j0], sd0, si0).wait()
        pltpu.async_copy(ztab.at[sd0.at[0]], gs0, sa0)
        pltpu.async_copy(ztab.at[sd0.at[1]], gd0, sb0)
        pltpu.make_async_copy(sd_hbm.at[w, j1], sd1, si1).wait()
        pltpu.async_copy(ztab.at[sd1.at[0]], gs1, sa1)
        pltpu.async_copy(ztab.at[sd1.at[1]], gd1, sb1)

        pltpu.make_async_copy(ztab.at[sd0.at[0]], gs0, sa0).wait()
        pltpu.make_async_copy(ztab.at[sd0.at[1]], gd0, sb0).wait()
        compute_chunk(j0, gs0, gd0)

        @pl.when(j0 + 2 < NCHUNK)
        def _():
            pltpu.async_copy(sd_hbm.at[w, j0 + 2], sd0, si0)

        pltpu.make_async_copy(ztab.at[sd1.at[0]], gs1, sa1).wait()
        pltpu.make_async_copy(ztab.at[sd1.at[1]], gd1, sb1).wait()
        compute_chunk(j1, gs1, gd1)

        @pl.when(j1 + 2 < NCHUNK)
        def _():
            pltpu.async_copy(sd_hbm.at[w, j1 + 2], sd1, si1)

        return 0

    lax.fori_loop(0, NCHUNK // 2, step, 0)


BR = 1280
GRID = NPAD // BR


def _rowspec(width):
    return pl.BlockSpec((BR, width), lambda i: (i, 0))


def _fullspec(shape):
    nd = len(shape)
    return pl.BlockSpec(shape, lambda i, _nd=nd: (0,) * _nd)


def _tc_prologue(deg0, deg1, x_pad, w_proj):
    def body(d0, d1, x, wp, z0, u1, disb, d2b):
        deg = d0[:, :1] + d1[:, :1] + 1.0
        dis = lax.rsqrt(deg)
        z = lax.dot_general(x[...], wp[...], (((1,), (1,)), ((), ())),
                            precision=_HIGH, preferred_element_type=jnp.float32)
        z0[...] = z
        u1[...] = z * dis
        disb[...] = jnp.broadcast_to(dis, (BR, F))
        d2b[...] = jnp.broadcast_to(dis * dis, (BR, F))

    outs = [jax.ShapeDtypeStruct((NPAD, F), jnp.float32) for _ in range(4)]
    return pl.pallas_call(
        body,
        grid=(GRID,),
        in_specs=[_rowspec(DW), _rowspec(DW), _rowspec(IN_DIM),
                  _fullspec((F, IN_DIM))],
        out_specs=[_rowspec(F)] * 4,
        out_shape=outs,
    )(deg0, deg1, x_pad, w_proj)


def _tc_pre_bn(a0, a1, disb, wmat, bias, width):
    def finish(p, br, pre, stats):
        i = pl.program_id(0)

        @pl.when(i == 0)
        def _():
            stats[...] = jnp.zeros_like(stats)

        p = p + br[...]
        pre[...] = p
        rows = i * BR + lax.broadcasted_iota(jnp.int32, (BR, 1), 0)
        pm = jnp.where(rows < N, p, 0.0)
        stats[0:1, :] += jnp.sum(pm, axis=0, keepdims=True)
        stats[1:2, :] += jnp.sum(pm * pm, axis=0, keepdims=True)

    if wmat is not None:
        def fn(a0r, a1r, dr, wr, br, pre, stats):
            p0 = dr[...] * (a0r[...] + a1r[...])
            p = lax.dot_general(p0, wr[...], (((1,), (1,)), ((), ())),
                                precision=_HIGH,
                                preferred_element_type=jnp.float32)
            finish(p, br, pre, stats)

        in_specs = [_rowspec(F), _rowspec(F), _rowspec(F),
                    _fullspec((width, F)), _fullspec((1, width))]
        args = [a0, a1, disb, wmat, bias]
    else:
        def fn(a0r, a1r, dr, br, pre, stats):
            finish(dr[...] * (a0r[...] + a1r[...]), br, pre, stats)

        in_specs = [_rowspec(F), _rowspec(F), _rowspec(F),
                    _fullspec((1, width))]
        args = [a0, a1, disb, bias]
    return pl.pallas_call(
        fn,
        grid=(GRID,),
        in_specs=in_specs,
        out_specs=[_rowspec(width), _fullspec((2, width))],
        out_shape=[jax.ShapeDtypeStruct((NPAD, width), jnp.float32),
                   jax.ShapeDtypeStruct((2, width), jnp.float32)],
    )(*args)


def _tc_bn_relu_proj(pre, stats, gamma, beta, wmat, disb, width):
    def body(pr, st, gr, br, wr, dr, u):
        m = st[0:1, :] / float(N)
        v = st[1:2, :] / float(N) - m * m
        h = (pr[...] - m) * lax.rsqrt(v + EPS) * gr[...] + br[...]
        h = jnp.maximum(h, 0.0)
        mm = lax.dot_general(h, wr[...], (((1,), (1,)), ((), ())),
                             precision=_HIGH, preferred_element_type=jnp.float32)
        u[...] = dr[...] * mm

    return pl.pallas_call(
        body,
        grid=(GRID,),
        in_specs=[_rowspec(width), _fullspec((2, width)), _fullspec((1, width)),
                  _fullspec((1, width)), _fullspec((F, width)), _rowspec(F)],
        out_specs=_rowspec(F),
        out_shape=jax.ShapeDtypeStruct((NPAD, F), jnp.float32),
    )(pre, stats, gamma, beta, wmat, disb)


def _tc_bn_relu_v0(pre, stats, gamma, beta, disb):
    def body(pr, st, gr, br, dr, v0, g):
        m = st[0:1, :] / float(N)
        v = st[1:2, :] / float(N) - m * m
        h = (pr[...] - m) * lax.rsqrt(v + EPS) * gr[...] + br[...]
        h = jnp.maximum(h, 0.0)
        vv = dr[...] * h
        v0[...] = vv
        g[...] = ALPHA * vv

    return pl.pallas_call(
        body,
        grid=(GRID,),
        in_specs=[_rowspec(F), _fullspec((2, F)), _fullspec((1, F)),
                  _fullspec((1, F)), _rowspec(F)],
        out_specs=[_rowspec(F)] * 2,
        out_shape=[jax.ShapeDtypeStruct((NPAD, F), jnp.float32)] * 2,
    )(pre, stats, gamma, beta, disb)


def _tc_appnp_stage(a0, a1, d2b, g):
    def body(a0r, a1r, d2r, gr, u):
        u[...] = (1.0 - ALPHA) * d2r[...] * (a0r[...] + a1r[...]) + gr[...]

    return pl.pallas_call(
        body,
        grid=(GRID,),
        in_specs=[_rowspec(F)] * 4,
        out_specs=_rowspec(F),
        out_shape=jax.ShapeDtypeStruct((NPAD, F), jnp.float32),
    )(a0, a1, d2b, g)


def _tc_finish(a0, a1, d2b, g, disb, z0):
    def body(a0r, a1r, d2r, gr, dr, z0r, zf):
        v = (1.0 - ALPHA) * d2r[...] * (a0r[...] + a1r[...]) + gr[...]
        zf[...] = v / dr[...] + z0r[...]

    return pl.pallas_call(
        body,
        grid=(GRID,),
        in_specs=[_rowspec(F)] * 6,
        out_specs=_rowspec(F),
        out_shape=jax.ShapeDtypeStruct((NPAD, F), jnp.float32),
    )(a0, a1, d2b, g, disb, z0)


def kernel(x, edge_index, W_proj, W1, b1, gamma1, beta1, W2, b2, gamma2, beta2):
    src = edge_index[0]
    dst = edge_index[1]

    pad_n = SLOTS - EPT
    w_ids = jnp.arange(NW, dtype=jnp.int32)[:, None]
    k_ids = jnp.arange(pad_n, dtype=jnp.int32)[None, :]
    pad_rows = N + (w_ids * 7 + k_ids) % (NPAD - N)
    srcp = jnp.concatenate([src.reshape(NW, EPT), pad_rows], axis=1)
    dstp = jnp.concatenate([dst.reshape(NW, EPT), pad_rows], axis=1)
    sdp = jnp.stack([srcp.reshape(NW, NCHUNK, CH),
                     dstp.reshape(NW, NCHUNK, CH)], axis=2)
    dstp_deg = dstp.reshape(NW, NCHD, CHD)

    x_pad = jnp.pad(x, ((0, NPAD - N), (0, 0)))
    znode = jnp.zeros((NPAD, F), jnp.float32)

    degs = _sc_degree(dstp_deg)
    z0, u1, disb, d2b = _tc_prologue(degs[0], degs[1], x_pad, W_proj)

    a0, a1 = _prop_id(sdp, znode, u1)
    pre1, stats1 = _tc_pre_bn(a0, a1, disb, W1, b1.reshape(1, HID), HID)
    u2 = _tc_bn_relu_proj(pre1, stats1, gamma1.reshape(1, HID),
                          beta1.reshape(1, HID), W2, disb, HID)

    a0, a1 = _prop_id(sdp, znode, u2)
    pre2, stats2 = _tc_pre_bn(a0, a1, disb, None, b2.reshape(1, F), F)
    v0, g = _tc_bn_relu_v0(pre2, stats2, gamma2.reshape(1, F),
                           beta2.reshape(1, F), disb)

    a0, a1 = _prop_id(sdp, znode, v0)
    for _ in range(K_PROP - 1):
        u = _tc_appnp_stage(a0, a1, d2b, g)
        a0, a1 = _prop_id(sdp, znode, u)

    z_final_pad = _tc_finish(a0, a1, d2b, g, disb, z0)

    dec = _sc_decode(sdp, z_final_pad)
    probs = dec.reshape(NW, SLOTS)[:, :EPT].reshape(E)
    return probs, z_final_pad[:N]
